# baseline (speedup 1.0000x reference)
import functools

import jax
import jax.numpy as jnp
from jax import lax
from jax.experimental import pallas as pl
from jax.experimental.pallas import tpu as pltpu
from jax.experimental.pallas import tpu_sc as plsc

BATCH = 16384
F_CAT = 26
F_CON = 13
D = 64
CAT_DIM = 100000
ROWS2 = F_CAT * CAT_DIM // 2
NC, NS, L = 2, 16, 16
NW = NC * NS
BW = BATCH // NW
NV = D // L
C = 64
NCH = BW // C
P_CAT = F_CAT // 2
P_CON = F_CON // 2
N_CAT_ITEMS = P_CAT * NCH
N_CON_ITEMS = P_CON * NCH
OUT_W = (F_CAT + F_CON) * D

_mesh = plsc.VectorSubcoreMesh(core_axis_name="c", subcore_axis_name="s")


@functools.partial(
    pl.kernel,
    out_type=jax.ShapeDtypeStruct((BATCH, OUT_W), jnp.float32),
    mesh=_mesh,
    scratch_types=[
        pltpu.VMEM((F_CAT, BW), jnp.int32),
        pltpu.VMEM((F_CON, BW), jnp.float32),
        pltpu.VMEM((F_CAT, D), jnp.float32),
        pltpu.VMEM((F_CON, D), jnp.float32),
        pltpu.VMEM((F_CON, D), jnp.float32),
        [pltpu.VMEM((C, 2 * D), jnp.float32) for _ in range(2)],
        [pltpu.VMEM((C, 2 * D), jnp.float32) for _ in range(2)],
        [pltpu.VMEM((C, 2 * D), jnp.float32) for _ in range(2)],
        [pltpu.VMEM((C, D), jnp.float32) for _ in range(2)],
        [pltpu.VMEM((C,), jnp.int32) for _ in range(2)],
        [pltpu.VMEM((C,), jnp.int32) for _ in range(2)],
        [pltpu.VMEM((C,), jnp.int32) for _ in range(2)],
        [pltpu.VMEM((C,), jnp.int32) for _ in range(2)],
        [pltpu.SemaphoreType.DMA for _ in range(2)],
        [pltpu.SemaphoreType.DMA for _ in range(2)],
        [pltpu.SemaphoreType.DMA for _ in range(2)],
        [pltpu.SemaphoreType.DMA for _ in range(2)],
    ],
    compiler_params=pltpu.CompilerParams(use_tc_tiling_on_sc=True),
)
def _fusion_tokenizer(cat_t_hbm, con_t_hbm, emb2_hbm, cat_bias_hbm,
                      num_w_hbm, num_b_hbm, out_hbm,
                      idx_all, con_all, bias_v, w_v, nb_v,
                      pbufA, pbufB, sbuf, tailbuf,
                      pidxA, pidxB, poffA, poffB,
                      gsemA, gsemB, ssem, tsem):
    wid = lax.axis_index("s") * NC + lax.axis_index("c")
    base = pl.multiple_of(wid * BW, BW)

    pltpu.sync_copy(cat_bias_hbm, bias_v)
    pltpu.sync_copy(num_w_hbm, w_v)
    pltpu.sync_copy(num_b_hbm, nb_v)
    pltpu.sync_copy(cat_t_hbm.at[:, pl.ds(base, BW)], idx_all)
    pltpu.sync_copy(con_t_hbm.at[:, pl.ds(base, BW)], con_all)

    def prep_and_fire(k, s):
        P = k // NCH
        c = k - P * NCH
        fA = 2 * P
        fB = fA + 1
        cb = c * C

        @pl.loop(0, C // L)
        def _mkidx(p):
            sl = pl.ds(cb + p * L, L)
            dl = pl.ds(p * L, L)
            tA = idx_all[fA, sl] + fA * CAT_DIM
            pidxA[s][dl] = tA >> 1
            poffA[s][dl] = (tA & 1) * D
            tB = idx_all[fB, sl] + fB * CAT_DIM
            pidxB[s][dl] = tB >> 1
            poffB[s][dl] = (tB & 1) * D

        pltpu.async_copy(emb2_hbm.at[pidxA[s]], pbufA[s], gsemA[s])
        pltpu.async_copy(emb2_hbm.at[pidxB[s]], pbufB[s], gsemB[s])

    def wait_gathers(s):
        pltpu.make_async_copy(emb2_hbm.at[pidxA[s]], pbufA[s], gsemA[s]).wait()
        pltpu.make_async_copy(emb2_hbm.at[pidxB[s]], pbufB[s], gsemB[s]).wait()

    def cat_store_dst(k, s):
        P = k // NCH
        c = k - P * NCH
        row = pl.multiple_of(base + c * C, 8)
        col = pl.multiple_of(P * 2 * D, 2 * D)
        return out_hbm.at[pl.ds(row, C), pl.ds(col, 2 * D)]

    def cat_compute(k, s):
        P = k // NCH
        fA = 2 * P
        fB = fA + 1
        bA = [bias_v[fA, pl.ds(q * L, L)] for q in range(NV)]
        bB = [bias_v[fB, pl.ds(q * L, L)] for q in range(NV)]

        @pl.loop(0, C // L)
        def _grp(g):
            pvA = poffA[s][pl.ds(g * L, L)]
            pvB = poffB[s][pl.ds(g * L, L)]
            for l in range(L):
                r = g * L + l
                selA = pvA[l] > 0
                selB = pvB[l] > 0
                for q in range(NV):
                    h0 = pbufA[s][r, pl.ds(q * L, L)]
                    h1 = pbufA[s][r, pl.ds(D + q * L, L)]
                    sbuf[s][r, pl.ds(q * L, L)] = (
                        jnp.where(selA, h1, h0) + bA[q])
                for q in range(NV):
                    h0 = pbufB[s][r, pl.ds(q * L, L)]
                    h1 = pbufB[s][r, pl.ds(D + q * L, L)]
                    sbuf[s][r, pl.ds(D + q * L, L)] = (
                        jnp.where(selB, h1, h0) + bB[q])

    prep_and_fire(0, 0)
    prep_and_fire(1, 1)

    @pl.loop(0, N_CAT_ITEMS, step=2)
    def _cat_ring(k0):
        for s in range(2):
            k = k0 + s
            wait_gathers(s)

            @pl.when(k >= 2)
            def _():
                pltpu.make_async_copy(sbuf[s], cat_store_dst(k - 2, s),
                                      ssem[s]).wait()

            cat_compute(k, s)
            pltpu.async_copy(sbuf[s], cat_store_dst(k, s), ssem[s])

            @pl.when(k + 2 < N_CAT_ITEMS)
            def _():
                prep_and_fire(k + 2, s)

    for s in range(2):
        k_last = N_CAT_ITEMS - 2 + s
        pltpu.make_async_copy(sbuf[s], cat_store_dst(k_last, s), ssem[s]).wait()

    def con_store_dst(k, s):
        Q = k // NCH
        c = k - Q * NCH
        row = pl.multiple_of(base + c * C, 8)
        col = pl.multiple_of(F_CAT * D + Q * 2 * D, 2 * D)
        return out_hbm.at[pl.ds(row, C), pl.ds(col, 2 * D)]

    def con_compute(k, s):
        Q = k // NCH
        c = k - Q * NCH
        jA = 2 * Q
        jB = jA + 1
        cb = c * C
        wA = [w_v[jA, pl.ds(q * L, L)] for q in range(NV)]
        wB = [w_v[jB, pl.ds(q * L, L)] for q in range(NV)]
        bA = [nb_v[jA, pl.ds(q * L, L)] for q in range(NV)]
        bB = [nb_v[jB, pl.ds(q * L, L)] for q in range(NV)]

        @pl.loop(0, C // L)
        def _grp(g):
            vA = con_all[jA, pl.ds(cb + g * L, L)]
            vB = con_all[jB, pl.ds(cb + g * L, L)]
            for l in range(L):
                r = g * L + l
                sA = vA[l]
                sB = vB[l]
                for q in range(NV):
                    sbuf[s][r, pl.ds(q * L, L)] = wA[q] * sA + bA[q]
                for q in range(NV):
                    sbuf[s][r, pl.ds(D + q * L, L)] = wB[q] * sB + bB[q]

    @pl.loop(0, N_CON_ITEMS, step=2)
    def _con_ring(k0):
        for s in range(2):
            k = k0 + s

            @pl.when(k >= 2)
            def _():
                pltpu.make_async_copy(sbuf[s], con_store_dst(k - 2, s),
                                      ssem[s]).wait()

            con_compute(k, s)
            pltpu.async_copy(sbuf[s], con_store_dst(k, s), ssem[s])

    for s in range(2):
        k_last = N_CON_ITEMS - 2 + s
        pltpu.make_async_copy(sbuf[s], con_store_dst(k_last, s), ssem[s]).wait()

    jT = F_CON - 1
    wT = [w_v[jT, pl.ds(q * L, L)] for q in range(NV)]
    bT = [nb_v[jT, pl.ds(q * L, L)] for q in range(NV)]

    def tail_dst(c, s):
        row = pl.multiple_of(base + c * C, 8)
        return out_hbm.at[pl.ds(row, C), pl.ds(OUT_W - D, D)]

    @pl.loop(0, NCH, step=2)
    def _tail_ring(c0):
        for s in range(2):
            c = c0 + s

            @pl.when(c >= 2)
            def _():
                pltpu.make_async_copy(tailbuf[s], tail_dst(c - 2, s),
                                      tsem[s]).wait()

            cb = c * C

            @pl.loop(0, C // L)
            def _grp(g):
                vT = con_all[jT, pl.ds(cb + g * L, L)]
                for l in range(L):
                    r = g * L + l
                    sT = vT[l]
                    for q in range(NV):
                        tailbuf[s][r, pl.ds(q * L, L)] = wT[q] * sT + bT[q]

            pltpu.async_copy(tailbuf[s], tail_dst(c, s), tsem[s])

    for s in range(2):
        c_last = NCH - 2 + s
        pltpu.make_async_copy(tailbuf[s], tail_dst(c_last, s), tsem[s]).wait()


def kernel(anchor_cat, anchor_con, emb_table, cat_bias, num_weight, num_bias):
    cat_t = anchor_cat.T
    con_t = anchor_con.T
    emb2 = emb_table.reshape(ROWS2, 2 * D)
    return _fusion_tokenizer(cat_t, con_t, emb2, cat_bias,
                             num_weight, num_bias)


# speedup vs baseline: 1.0508x; 1.0508x over previous
import functools

import jax
import jax.numpy as jnp
from jax import lax
from jax.experimental import pallas as pl
from jax.experimental.pallas import tpu as pltpu
from jax.experimental.pallas import tpu_sc as plsc

BATCH = 16384
F_CAT = 26
F_CON = 13
D = 64
CAT_DIM = 100000
NC, NS, L = 2, 16, 16
NW = NC * NS
BW = BATCH // NW
NV = D // L
NSLOT = 3
OUT_W = (F_CAT + F_CON) * D

_mesh = plsc.VectorSubcoreMesh(core_axis_name="c", subcore_axis_name="s")


@functools.partial(
    pl.kernel,
    out_type=jax.ShapeDtypeStruct((BATCH, OUT_W), jnp.float32),
    mesh=_mesh,
    scratch_types=[
        pltpu.VMEM((F_CAT, BW), jnp.int32),
        pltpu.VMEM((F_CON, BW), jnp.float32),
        [pltpu.VMEM((BW, D), jnp.float32) for _ in range(NSLOT)],
        pltpu.VMEM((F_CAT, D), jnp.float32),
        pltpu.VMEM((F_CON, D), jnp.float32),
        pltpu.VMEM((F_CON, D), jnp.float32),
        [pltpu.SemaphoreType.DMA for _ in range(NSLOT)],
        [pltpu.SemaphoreType.DMA for _ in range(NSLOT)],
        pltpu.SemaphoreType.DMA,
    ],
    compiler_params=pltpu.CompilerParams(use_tc_tiling_on_sc=False),
)
def _fusion_tokenizer(cat_hbm, con_hbm, emb_hbm, cat_bias_hbm,
                      num_w_hbm, num_b_hbm, out_hbm,
                      idx_all, con_all, rows, bias_v, w_v, nb_v,
                      gsem, ssem, stsem):
    wid = lax.axis_index("s") * NC + lax.axis_index("c")
    base = pl.multiple_of(wid * BW, BW)

    pltpu.async_copy(cat_hbm.at[:, pl.ds(base, BW)], idx_all, stsem)
    pltpu.sync_copy(con_hbm.at[:, pl.ds(base, BW)], con_all)
    pltpu.sync_copy(cat_bias_hbm, bias_v)
    pltpu.sync_copy(num_w_hbm, w_v)
    pltpu.sync_copy(num_b_hbm, nb_v)
    pltpu.make_async_copy(cat_hbm.at[:, pl.ds(base, BW)], idx_all, stsem).wait()

    @pl.loop(0, F_CAT)
    def _field_off(f):
        off = f * CAT_DIM

        @pl.loop(0, BW // L, unroll=4)
        def _add_off(p):
            idx_all[f, pl.ds(p * L, L)] = idx_all[f, pl.ds(p * L, L)] + off

    def fire_gather(f):
        s = f % NSLOT
        pltpu.async_copy(emb_hbm.at[idx_all.at[f]], rows[s], gsem[s])

    def wait_gather(f):
        s = f % NSLOT
        pltpu.make_async_copy(emb_hbm.at[idx_all.at[f]], rows[s], gsem[s]).wait()

    def store_dst(f):
        return out_hbm.at[pl.ds(base, BW), pl.ds(f * D, D)]

    def fire_store(f):
        s = f % NSLOT
        pltpu.async_copy(rows[s], store_dst(f), ssem[s])

    def wait_store(f):
        s = f % NSLOT
        pltpu.make_async_copy(rows[s], store_dst(f), ssem[s]).wait()

    for f in range(NSLOT):
        fire_gather(f)

    for f in range(F_CAT):
        s = f % NSLOT
        wait_gather(f)
        bregs = [bias_v[f, pl.ds(q * L, L)] for q in range(NV)]

        @pl.loop(0, BW, unroll=4)
        def _bias_add(r):
            for q in range(NV):
                rows[s][r, pl.ds(q * L, L)] = rows[s][r, pl.ds(q * L, L)] + bregs[q]

        fire_store(f)
        if f + NSLOT < F_CAT:
            wait_store(f)
            fire_gather(f + NSLOT)

    for j in range(F_CON):
        f = F_CAT + j
        s = f % NSLOT
        wait_store(f - NSLOT)
        wregs = [w_v[j, pl.ds(q * L, L)] for q in range(NV)]
        bregs = [nb_v[j, pl.ds(q * L, L)] for q in range(NV)]

        @pl.loop(0, BW // L)
        def _rowgroup(g):
            v16 = con_all[j, pl.ds(g * L, L)]
            for l in range(L):
                sc = v16[l]
                r = g * L + l
                for q in range(NV):
                    rows[s][r, pl.ds(q * L, L)] = wregs[q] * sc + bregs[q]

        fire_store(f)

    for f in range(F_CAT + F_CON - NSLOT, F_CAT + F_CON):
        wait_store(f)


def kernel(anchor_cat, anchor_con, emb_table, cat_bias, num_weight, num_bias):
    cat_t, con_t = jax.lax.optimization_barrier(
        (anchor_cat.T, anchor_con.T))
    return _fusion_tokenizer(cat_t, con_t, emb_table, cat_bias,
                             num_weight, num_bias)


# speedup vs baseline: 1.0530x; 1.0021x over previous
import functools

import jax
import jax.numpy as jnp
from jax import lax
from jax.experimental import pallas as pl
from jax.experimental.pallas import tpu as pltpu
from jax.experimental.pallas import tpu_sc as plsc

BATCH = 16384
F_CAT = 26
F_CON = 13
D = 64
CAT_DIM = 100000
NC, NS, L = 2, 16, 16
NW = NC * NS
BW = BATCH // NW
NV = D // L
NSLOT = 3
OUT_W = (F_CAT + F_CON) * D

_mesh = plsc.VectorSubcoreMesh(core_axis_name="c", subcore_axis_name="s")


@functools.partial(
    pl.kernel,
    out_type=jax.ShapeDtypeStruct((BATCH, OUT_W), jnp.float32),
    mesh=_mesh,
    scratch_types=[
        pltpu.VMEM((F_CAT, BW), jnp.int32),
        pltpu.VMEM((F_CON, BW), jnp.float32),
        [pltpu.VMEM((BW, D), jnp.float32) for _ in range(NSLOT)],
        pltpu.VMEM((F_CAT, D), jnp.float32),
        pltpu.VMEM((F_CON, D), jnp.float32),
        pltpu.VMEM((F_CON, D), jnp.float32),
        [pltpu.SemaphoreType.DMA for _ in range(NSLOT)],
        [pltpu.SemaphoreType.DMA for _ in range(NSLOT)],
        pltpu.SemaphoreType.DMA,
    ],
    compiler_params=pltpu.CompilerParams(use_tc_tiling_on_sc=False),
)
def _fusion_tokenizer(cat_hbm, con_hbm, emb_hbm, cat_bias_hbm,
                      num_w_hbm, num_b_hbm, out_hbm,
                      idx_all, con_all, rows, bias_v, w_v, nb_v,
                      gsem, ssem, stsem):
    wid = lax.axis_index("s") * NC + lax.axis_index("c")
    base = pl.multiple_of(wid * BW, BW)

    pltpu.async_copy(cat_hbm.at[:, pl.ds(base, BW)], idx_all, stsem)
    pltpu.sync_copy(con_hbm.at[:, pl.ds(base, BW)], con_all)
    pltpu.sync_copy(cat_bias_hbm, bias_v)
    pltpu.sync_copy(num_w_hbm, w_v)
    pltpu.sync_copy(num_b_hbm, nb_v)
    pltpu.make_async_copy(cat_hbm.at[:, pl.ds(base, BW)], idx_all, stsem).wait()

    @pl.loop(0, F_CAT)
    def _field_off(f):
        off = f * CAT_DIM

        @pl.loop(0, BW // L, unroll=4)
        def _add_off(p):
            idx_all[f, pl.ds(p * L, L)] = idx_all[f, pl.ds(p * L, L)] + off

    def fire_gather(f):
        s = f % NSLOT
        pltpu.async_copy(emb_hbm.at[idx_all.at[f]], rows[s], gsem[s])

    def wait_gather(f):
        s = f % NSLOT
        pltpu.make_async_copy(emb_hbm.at[idx_all.at[f]], rows[s], gsem[s]).wait()

    def store_dst(f):
        return out_hbm.at[pl.ds(base, BW), pl.ds(f * D, D)]

    def fire_store(f):
        s = f % NSLOT
        pltpu.async_copy(rows[s], store_dst(f), ssem[s])

    def wait_store(f):
        s = f % NSLOT
        pltpu.make_async_copy(rows[s], store_dst(f), ssem[s]).wait()

    for f in range(NSLOT):
        fire_gather(f)

    for f in range(F_CAT):
        s = f % NSLOT
        wait_gather(f)
        bregs = [bias_v[f, pl.ds(q * L, L)] for q in range(NV)]

        @pl.loop(0, BW, unroll=4)
        def _bias_add(r):
            for q in range(NV):
                rows[s][r, pl.ds(q * L, L)] = rows[s][r, pl.ds(q * L, L)] + bregs[q]

        fire_store(f)
        if f + NSLOT < F_CAT:
            wait_store(f)
            fire_gather(f + NSLOT)

    for j in range(F_CON):
        f = F_CAT + j
        s = f % NSLOT
        wait_store(f - NSLOT)
        wregs = [w_v[j, pl.ds(q * L, L)] for q in range(NV)]
        bregs = [nb_v[j, pl.ds(q * L, L)] for q in range(NV)]

        @pl.loop(0, BW // L)
        def _rowgroup(g):
            v16 = con_all[j, pl.ds(g * L, L)]
            for l in range(L):
                sc = v16[l]
                r = g * L + l
                for q in range(NV):
                    rows[s][r, pl.ds(q * L, L)] = wregs[q] * sc + bregs[q]

        fire_store(f)

    for f in range(F_CAT + F_CON - NSLOT, F_CAT + F_CON):
        wait_store(f)


def kernel(anchor_cat, anchor_con, emb_table, cat_bias, num_weight, num_bias):
    cat_t = anchor_cat.T
    con_t = anchor_con.T
    one = (anchor_cat[0, 0] * 0 + 1).astype(jnp.float32)
    emb3 = emb_table * one
    return _fusion_tokenizer(cat_t, con_t, emb3, cat_bias,
                             num_weight, num_bias)


# speedup vs baseline: 1.7286x; 1.6416x over previous
import functools

import jax
import jax.numpy as jnp
from jax import lax
from jax.experimental import pallas as pl
from jax.experimental.pallas import tpu as pltpu
from jax.experimental.pallas import tpu_sc as plsc

BATCH = 16384
F_CAT = 26
F_CON = 13
D = 64
CAT_DIM = 100000
NC, NS, L = 2, 16, 16
NW = NC * NS
BW = BATCH // NW
NV = D // L
C = 64
NCH = BW // C
P_CAT = F_CAT // 2
P_CON = F_CON // 2
N_CAT_ITEMS = P_CAT * NCH
N_CON_ITEMS = P_CON * NCH
OUT_W = (F_CAT + F_CON) * D

_mesh = plsc.VectorSubcoreMesh(core_axis_name="c", subcore_axis_name="s")


@functools.partial(
    pl.kernel,
    out_type=jax.ShapeDtypeStruct((BATCH, OUT_W), jnp.float32),
    mesh=_mesh,
    scratch_types=[
        pltpu.VMEM((F_CAT, BW), jnp.int32),
        pltpu.VMEM((F_CON, BW), jnp.float32),
        pltpu.VMEM((F_CAT, D), jnp.float32),
        pltpu.VMEM((F_CON, D), jnp.float32),
        pltpu.VMEM((F_CON, D), jnp.float32),
        [pltpu.VMEM((C, D), jnp.float32) for _ in range(2)],
        [pltpu.VMEM((C, D), jnp.float32) for _ in range(2)],
        [pltpu.VMEM((C, 2 * D), jnp.float32) for _ in range(2)],
        [pltpu.VMEM((C, D), jnp.float32) for _ in range(2)],
        [pltpu.SemaphoreType.DMA for _ in range(2)],
        [pltpu.SemaphoreType.DMA for _ in range(2)],
        [pltpu.SemaphoreType.DMA for _ in range(2)],
        [pltpu.SemaphoreType.DMA for _ in range(2)],
    ],
    compiler_params=pltpu.CompilerParams(use_tc_tiling_on_sc=True),
)
def _fusion_tokenizer(cat_t_hbm, con_t_hbm, emb_hbm, cat_bias_hbm,
                      num_w_hbm, num_b_hbm, out_hbm,
                      idx_all, con_all, bias_v, w_v, nb_v,
                      rbufA, rbufB, sbuf, tailbuf,
                      gsemA, gsemB, ssem, tsem):
    wid = lax.axis_index("s") * NC + lax.axis_index("c")
    base = pl.multiple_of(wid * BW, BW)

    pltpu.sync_copy(cat_bias_hbm, bias_v)
    pltpu.sync_copy(num_w_hbm, w_v)
    pltpu.sync_copy(num_b_hbm, nb_v)
    pltpu.sync_copy(cat_t_hbm.at[:, pl.ds(base, BW)], idx_all)
    pltpu.sync_copy(con_t_hbm.at[:, pl.ds(base, BW)], con_all)

    def prep_and_fire(k, s):
        P = k // NCH
        c = k - P * NCH
        fA = 2 * P
        fB = fA + 1
        cb = c * C

        @pl.loop(0, C // L)
        def _fire(g):
            sl = pl.ds(cb + g * L, L)
            vA = idx_all[fA, sl] + fA * CAT_DIM
            vB = idx_all[fB, sl] + fB * CAT_DIM
            for l in range(L):
                r = g * L + l
                pltpu.async_copy(emb_hbm.at[pl.ds(vA[l], 1), :],
                                 rbufA[s].at[pl.ds(r, 1), :], gsemA[s])
                pltpu.async_copy(emb_hbm.at[pl.ds(vB[l], 1), :],
                                 rbufB[s].at[pl.ds(r, 1), :], gsemB[s])

    def wait_gathers(s):
        pltpu.make_async_copy(emb_hbm.at[pl.ds(0, C), :], rbufA[s],
                              gsemA[s]).wait()
        pltpu.make_async_copy(emb_hbm.at[pl.ds(0, C), :], rbufB[s],
                              gsemB[s]).wait()

    def cat_store_dst(k, s):
        P = k // NCH
        c = k - P * NCH
        row = pl.multiple_of(base + c * C, 8)
        col = pl.multiple_of(P * 2 * D, 2 * D)
        return out_hbm.at[pl.ds(row, C), pl.ds(col, 2 * D)]

    def cat_compute(k, s):
        P = k // NCH
        fA = 2 * P
        fB = fA + 1
        bA = [bias_v[fA, pl.ds(q * L, L)] for q in range(NV)]
        bB = [bias_v[fB, pl.ds(q * L, L)] for q in range(NV)]

        @pl.loop(0, C)
        def _row(r):
            for q in range(NV):
                sbuf[s][r, pl.ds(q * L, L)] = rbufA[s][r, pl.ds(q * L, L)] + bA[q]
            for q in range(NV):
                sbuf[s][r, pl.ds(D + q * L, L)] = rbufB[s][r, pl.ds(q * L, L)] + bB[q]

    prep_and_fire(0, 0)
    prep_and_fire(1, 1)

    @pl.loop(0, N_CAT_ITEMS, step=2)
    def _cat_ring(k0):
        for s in range(2):
            k = k0 + s
            wait_gathers(s)

            @pl.when(k >= 2)
            def _():
                pltpu.make_async_copy(sbuf[s], cat_store_dst(k - 2, s),
                                      ssem[s]).wait()

            cat_compute(k, s)
            pltpu.async_copy(sbuf[s], cat_store_dst(k, s), ssem[s])

            @pl.when(k + 2 < N_CAT_ITEMS)
            def _():
                prep_and_fire(k + 2, s)

    for s in range(2):
        k_last = N_CAT_ITEMS - 2 + s
        pltpu.make_async_copy(sbuf[s], cat_store_dst(k_last, s), ssem[s]).wait()

    def con_store_dst(k, s):
        Q = k // NCH
        c = k - Q * NCH
        row = pl.multiple_of(base + c * C, 8)
        col = pl.multiple_of(F_CAT * D + Q * 2 * D, 2 * D)
        return out_hbm.at[pl.ds(row, C), pl.ds(col, 2 * D)]

    def con_compute(k, s):
        Q = k // NCH
        c = k - Q * NCH
        jA = 2 * Q
        jB = jA + 1
        cb = c * C
        wA = [w_v[jA, pl.ds(q * L, L)] for q in range(NV)]
        wB = [w_v[jB, pl.ds(q * L, L)] for q in range(NV)]
        bA = [nb_v[jA, pl.ds(q * L, L)] for q in range(NV)]
        bB = [nb_v[jB, pl.ds(q * L, L)] for q in range(NV)]

        @pl.loop(0, C // L)
        def _grp(g):
            vA = con_all[jA, pl.ds(cb + g * L, L)]
            vB = con_all[jB, pl.ds(cb + g * L, L)]
            for l in range(L):
                r = g * L + l
                sA = vA[l]
                sB = vB[l]
                for q in range(NV):
                    sbuf[s][r, pl.ds(q * L, L)] = wA[q] * sA + bA[q]
                for q in range(NV):
                    sbuf[s][r, pl.ds(D + q * L, L)] = wB[q] * sB + bB[q]

    @pl.loop(0, N_CON_ITEMS, step=2)
    def _con_ring(k0):
        for s in range(2):
            k = k0 + s

            @pl.when(k >= 2)
            def _():
                pltpu.make_async_copy(sbuf[s], con_store_dst(k - 2, s),
                                      ssem[s]).wait()

            con_compute(k, s)
            pltpu.async_copy(sbuf[s], con_store_dst(k, s), ssem[s])

    for s in range(2):
        k_last = N_CON_ITEMS - 2 + s
        pltpu.make_async_copy(sbuf[s], con_store_dst(k_last, s), ssem[s]).wait()

    jT = F_CON - 1
    wT = [w_v[jT, pl.ds(q * L, L)] for q in range(NV)]
    bT = [nb_v[jT, pl.ds(q * L, L)] for q in range(NV)]

    def tail_dst(c, s):
        row = pl.multiple_of(base + c * C, 8)
        return out_hbm.at[pl.ds(row, C), pl.ds(OUT_W - D, D)]

    @pl.loop(0, NCH, step=2)
    def _tail_ring(c0):
        for s in range(2):
            c = c0 + s

            @pl.when(c >= 2)
            def _():
                pltpu.make_async_copy(tailbuf[s], tail_dst(c - 2, s),
                                      tsem[s]).wait()

            cb = c * C

            @pl.loop(0, C // L)
            def _grp(g):
                vT = con_all[jT, pl.ds(cb + g * L, L)]
                for l in range(L):
                    r = g * L + l
                    sT = vT[l]
                    for q in range(NV):
                        tailbuf[s][r, pl.ds(q * L, L)] = wT[q] * sT + bT[q]

            pltpu.async_copy(tailbuf[s], tail_dst(c, s), tsem[s])

    for s in range(2):
        c_last = NCH - 2 + s
        pltpu.make_async_copy(tailbuf[s], tail_dst(c_last, s), tsem[s]).wait()


def kernel(anchor_cat, anchor_con, emb_table, cat_bias, num_weight, num_bias):
    cat_t = anchor_cat.T
    con_t = anchor_con.T
    return _fusion_tokenizer(cat_t, con_t, emb_table, cat_bias,
                             num_weight, num_bias)
